# baseline (speedup 1.0000x reference)
import functools

import jax
import jax.numpy as jnp
from jax import lax
from jax.experimental import pallas as pl
from jax.experimental.pallas import tpu as pltpu
from jax.experimental.pallas import tpu_sc as plsc

BATCH = 4096
SEQ = 200
HIDDEN = 128
LANES = 16

NW = 32
ROWS = BATCH * SEQ
ROWS_PER_W = ROWS // NW
CHUNK = 128
NCHUNK = ROWS_PER_W // CHUNK
POS_EXT = SEQ + CHUNK
NBUF = 5


def _build():
    mesh = plsc.VectorSubcoreMesh(core_axis_name="c", subcore_axis_name="s")

    @functools.partial(
        pl.kernel,
        out_type=jax.ShapeDtypeStruct((ROWS, HIDDEN), jnp.float32),
        mesh=mesh,
        scratch_types=[
            pltpu.VMEM((POS_EXT, HIDDEN), jnp.float32),
        ] + [pltpu.VMEM((CHUNK, HIDDEN), jnp.float32) for _ in range(NBUF)]
          + [pltpu.VMEM((CHUNK,), jnp.int32) for _ in range(NBUF)]
          + [pltpu.SemaphoreType.DMA for _ in range(3 * NBUF)],
    )
    def emb_kernel(ids_hbm, tok_hbm, pos_hbm, out_hbm, pos_v, *bs):
        bufs = bs[:NBUF]
        ibufs = bs[NBUF:2 * NBUF]
        gsem = bs[2 * NBUF:3 * NBUF]
        osem = bs[3 * NBUF:4 * NBUF]
        isem = bs[4 * NBUF:5 * NBUF]

        wid = lax.axis_index("s") * 2 + lax.axis_index("c")
        base = wid * ROWS_PER_W

        pltpu.sync_copy(pos_hbm, pos_v)

        def idx_src(j):
            return ids_hbm.at[pl.ds(base + j * CHUNK, CHUNK)]

        def fire_idx(j, s):
            pltpu.async_copy(idx_src(j), ibufs[s], isem[s])

        def wait_idx(j, s):
            pltpu.make_async_copy(idx_src(j), ibufs[s], isem[s]).wait()

        def fire_gather(s):
            pltpu.async_copy(tok_hbm.at[ibufs[s]], bufs[s], gsem[s])

        def wait_gather(s):
            pltpu.make_async_copy(tok_hbm.at[ibufs[s]], bufs[s],
                                  gsem[s]).wait()

        for s in range(NBUF):
            fire_idx(s, s)
        for s in range(NBUF):
            wait_idx(s, s)
            fire_gather(s)

        @pl.loop(0, NCHUNK, step=NBUF)
        def group(jb):
            for s in range(NBUF):
                j = jb + s
                wait_gather(s)
                nxt = j + NBUF

                @pl.when(nxt < NCHUNK)
                def _():
                    fire_idx(nxt, s)

                start = lax.rem(j * CHUNK, SEQ)

                @pl.loop(0, CHUNK, unroll=8)
                def row(r):
                    for g in range(HIDDEN // LANES):
                        vec = pos_v[start + r, pl.ds(g * LANES, LANES)]
                        plsc.addupdate(
                            bufs[s].at[r, pl.ds(g * LANES, LANES)], vec)

                dst = out_hbm.at[pl.ds(base + j * CHUNK, CHUNK)]
                pltpu.async_copy(bufs[s], dst, osem[s])

                @pl.when(nxt < NCHUNK)
                def _():
                    pltpu.make_async_copy(bufs[s], dst, osem[s]).wait()
                    wait_idx(nxt, s)
                    fire_gather(s)

        for s in range(NBUF):
            pltpu.make_async_copy(
                bufs[s], out_hbm.at[pl.ds(base, CHUNK)], osem[s]).wait()

    return emb_kernel


_emb = _build()


def kernel(input_ids, token_embedding, position_embedding):
    ids = input_ids.reshape(ROWS).astype(jnp.int32)
    pos_ext = jnp.concatenate(
        [position_embedding, position_embedding[:CHUNK]], axis=0)
    out = _emb(ids, token_embedding, pos_ext)
    return out.reshape(BATCH, SEQ, HIDDEN)


# speedup vs baseline: 2.4529x; 2.4529x over previous
import functools

import jax
import jax.numpy as jnp
from jax import lax
from jax.experimental import pallas as pl
from jax.experimental.pallas import tpu as pltpu
from jax.experimental.pallas import tpu_sc as plsc

BATCH = 4096
SEQ = 200
HIDDEN = 128
LANES = 16

NW = 32
ROWS = BATCH * SEQ
ROWS_PER_W = ROWS // NW
SEQ_PER_W = ROWS_PER_W // SEQ
R_A, R_B = 104, 96
DEPTH = 2


def _build():
    mesh = plsc.VectorSubcoreMesh(core_axis_name="c", subcore_axis_name="s")

    @functools.partial(
        pl.kernel,
        out_type=jax.ShapeDtypeStruct((ROWS, HIDDEN), jnp.float32),
        mesh=mesh,
        scratch_types=[
            pltpu.VMEM((ROWS_PER_W,), jnp.int32),
            pltpu.VMEM((SEQ, HIDDEN), jnp.float32),
        ] + [pltpu.VMEM((R_A, HIDDEN), jnp.float32) for _ in range(DEPTH)]
          + [pltpu.VMEM((R_B, HIDDEN), jnp.float32) for _ in range(DEPTH)]
          + [pltpu.SemaphoreType.DMA for _ in range(4 * DEPTH)],
    )
    def emb_kernel(ids_hbm, tok_hbm, pos_hbm, out_hbm, idx_v, pos_v, *bs):
        bufs = {"A": bs[:DEPTH], "B": bs[DEPTH:2 * DEPTH]}
        gsem = {"A": bs[2 * DEPTH:3 * DEPTH], "B": bs[3 * DEPTH:4 * DEPTH]}
        osem = {"A": bs[4 * DEPTH:5 * DEPTH], "B": bs[5 * DEPTH:6 * DEPTH]}
        rows = {"A": R_A, "B": R_B}
        off = {"A": 0, "B": R_A}

        wid = lax.axis_index("s") * 2 + lax.axis_index("c")
        base = wid * ROWS_PER_W

        pltpu.sync_copy(ids_hbm.at[wid], idx_v)
        pltpu.sync_copy(pos_hbm, pos_v)

        def gparts(q, kind, d):
            src = tok_hbm.at[idx_v.at[pl.ds(q * SEQ + off[kind], rows[kind])]]
            return src, bufs[kind][d], gsem[kind][d]

        def process(q, kind, d):
            pltpu.make_async_copy(*gparts(q, kind, d)).wait()

            @pl.loop(0, rows[kind], unroll=8)
            def row(r):
                for g in range(HIDDEN // LANES):
                    sl = pl.ds(g * LANES, LANES)
                    plsc.addupdate(bufs[kind][d].at[r, sl],
                                   pos_v[off[kind] + r, sl])

            dst = out_hbm.at[pl.ds(base + q * SEQ + off[kind], rows[kind])]
            pltpu.async_copy(bufs[kind][d], dst, osem[kind][d])

            @pl.when(q + DEPTH < SEQ_PER_W)
            def _():
                pltpu.make_async_copy(bufs[kind][d], dst,
                                      osem[kind][d]).wait()
                pltpu.async_copy(*gparts(q + DEPTH, kind, d))

        for d in range(DEPTH):
            for kind in ("A", "B"):
                pltpu.async_copy(*gparts(d, kind, d))

        @pl.loop(0, SEQ_PER_W, step=DEPTH)
        def group(qb):
            for d in range(DEPTH):
                for kind in ("A", "B"):
                    process(qb + d, kind, d)

        for d in range(DEPTH):
            for kind in ("A", "B"):
                pltpu.make_async_copy(
                    bufs[kind][d],
                    out_hbm.at[pl.ds(base + off[kind], rows[kind])],
                    osem[kind][d]).wait()

    return emb_kernel


_emb = _build()


def kernel(input_ids, token_embedding, position_embedding):
    ids = input_ids.reshape(NW, ROWS_PER_W).astype(jnp.int32)
    out = _emb(ids, token_embedding, position_embedding)
    return out.reshape(BATCH, SEQ, HIDDEN)


# speedup vs baseline: 2.6454x; 1.0785x over previous
import functools

import jax
import jax.numpy as jnp
from jax import lax
from jax.experimental import pallas as pl
from jax.experimental.pallas import tpu as pltpu
from jax.experimental.pallas import tpu_sc as plsc

BATCH = 4096
SEQ = 200
HIDDEN = 128

NW = 32
ROWS = BATCH * SEQ
ROWS_PER_W = ROWS // NW
SEQ_PER_W = ROWS_PER_W // SEQ
R_A, R_B = 104, 96
DEPTH = 2


def _build():
    mesh = plsc.VectorSubcoreMesh(core_axis_name="c", subcore_axis_name="s")

    @functools.partial(
        pl.kernel,
        out_type=jax.ShapeDtypeStruct((ROWS, HIDDEN), jnp.float32),
        mesh=mesh,
        scratch_types=[
            pltpu.VMEM((ROWS_PER_W,), jnp.int32),
            pltpu.VMEM_SHARED((SEQ, HIDDEN), jnp.float32),
        ] + [pltpu.VMEM((R_A, HIDDEN), jnp.float32) for _ in range(DEPTH)]
          + [pltpu.VMEM((R_B, HIDDEN), jnp.float32) for _ in range(DEPTH)]
          + [pltpu.SemaphoreType.DMA for _ in range(6 * DEPTH)],
    )
    def emb_kernel(ids_hbm, tok_hbm, pos_hbm, out_hbm, idx_v, pos_v, *bs):
        bufs = {"A": bs[:DEPTH], "B": bs[DEPTH:2 * DEPTH]}
        gsem = {"A": bs[2 * DEPTH:3 * DEPTH], "B": bs[3 * DEPTH:4 * DEPTH]}
        osem = {"A": bs[4 * DEPTH:5 * DEPTH], "B": bs[5 * DEPTH:6 * DEPTH]}
        psem = {"A": bs[6 * DEPTH:7 * DEPTH], "B": bs[7 * DEPTH:8 * DEPTH]}
        rows = {"A": R_A, "B": R_B}
        off = {"A": 0, "B": R_A}

        wid = lax.axis_index("s") * 2 + lax.axis_index("c")
        base = wid * ROWS_PER_W

        pltpu.sync_copy(ids_hbm.at[wid], idx_v)

        @pl.when(lax.axis_index("s") == 0)
        def _():
            pltpu.sync_copy(pos_hbm, pos_v)

        plsc.subcore_barrier()

        def pparts(kind, d):
            return (pos_v.at[pl.ds(off[kind], rows[kind])], bufs[kind][d],
                    psem[kind][d])

        def gparts(q, kind, d):
            src = tok_hbm.at[idx_v.at[pl.ds(q * SEQ + off[kind], rows[kind])]]
            return src, bufs[kind][d], gsem[kind][d]

        def fire_chain(q, kind, d):
            pltpu.async_copy(*pparts(kind, d))
            pltpu.make_async_copy(*pparts(kind, d)).wait()
            pltpu.async_copy(*gparts(q, kind, d), add=True)

        def process(q, kind, d):
            pltpu.make_async_copy(*gparts(q, kind, d)).wait()
            dst = out_hbm.at[pl.ds(base + q * SEQ + off[kind], rows[kind])]
            pltpu.async_copy(bufs[kind][d], dst, osem[kind][d])

            @pl.when(q + DEPTH < SEQ_PER_W)
            def _():
                pltpu.make_async_copy(bufs[kind][d], dst,
                                      osem[kind][d]).wait()
                fire_chain(q + DEPTH, kind, d)

        for d in range(DEPTH):
            for kind in ("A", "B"):
                fire_chain(d, kind, d)

        @pl.loop(0, SEQ_PER_W, step=DEPTH)
        def group(qb):
            for d in range(DEPTH):
                for kind in ("A", "B"):
                    process(qb + d, kind, d)

        for d in range(DEPTH):
            for kind in ("A", "B"):
                pltpu.make_async_copy(
                    bufs[kind][d],
                    out_hbm.at[pl.ds(base + off[kind], rows[kind])],
                    osem[kind][d]).wait()

    return emb_kernel


_emb = _build()


def kernel(input_ids, token_embedding, position_embedding):
    ids = input_ids.reshape(NW, ROWS_PER_W).astype(jnp.int32)
    out = _emb(ids, token_embedding, position_embedding)
    return out.reshape(BATCH, SEQ, HIDDEN)
